# baseline (speedup 1.0000x reference)
import jax
import jax.numpy as jnp
from jax import lax
from jax.experimental import pallas as pl
from jax.experimental.pallas import tpu as pltpu
from jax.experimental.pallas import tpu_sc as plsc

B = 16384
EMB = 128
NC = 2
NS = 16
NW = NC * NS
BPW = B // NW
CH = 128
NCH = BPW // CH
GRP = CH // 16


def _mf_body(x_hbm, w_hbm, h_hbm, out_hbm,
             xv, uid_v, iid_v, wb0, wb1, hb0, hb1, out_v,
             sw0, sw1, sh0, sh1):
    wid = lax.axis_index("s") * NC + lax.axis_index("c")
    base = wid * BPW

    lane = lax.iota(jnp.int32, 16)
    gmode = "promise_in_bounds"

    pltpu.sync_copy(x_hbm.at[pl.ds(base * 2, 2 * BPW)], xv)
    perm_u = (2 * lane) & 15
    perm_i = perm_u + 1
    lo_half = lane < 8
    for c in range(NCH):
        for g in range(GRP):
            off = 2 * (c * CH + g * 16)
            a = xv[pl.ds(off, 16)]
            b = xv[pl.ds(off + 16, 16)]
            au = a.at[perm_u].get(mode=gmode)
            bu = b.at[perm_u].get(mode=gmode)
            ai = a.at[perm_i].get(mode=gmode)
            bi = b.at[perm_i].get(mode=gmode)
            uid_v[c, pl.ds(g * 16, 16)] = jnp.where(lo_half, au, bu)
            iid_v[c, pl.ds(g * 16, 16)] = jnp.where(lo_half, ai, bi)

    wbufs = (wb0, wb1)
    hbufs = (hb0, hb1)
    wsems = (sw0, sw1)
    hsems = (sh0, sh1)

    def start(c):
        slot = c % 2
        cw = pltpu.async_copy(w_hbm.at[uid_v.at[c]], wbufs[slot], wsems[slot])
        chh = pltpu.async_copy(h_hbm.at[iid_v.at[c]], hbufs[slot], hsems[slot])
        return cw, chh

    merge_consts = [((lane & s) != 0, lane ^ s) for s in (1, 2, 4, 8)]

    inflight = {0: start(0)}
    for c in range(NCH):
        if c + 1 < NCH:
            inflight[c + 1] = start(c + 1)
        for h in inflight.pop(c):
            h.wait()
        slot = c % 2
        wref = wbufs[slot]
        href = hbufs[slot]

        def group_body(g, _, wref=wref, href=href, c=c):
            row0 = g * 16
            vecs = []
            for r in range(16):
                row = row0 + r
                ps = []
                for j in range(EMB // 16):
                    w = wref[row, pl.ds(j * 16, 16)]
                    h = href[row, pl.ds(j * 16, 16)]
                    ps.append(w * h)
                while len(ps) > 1:
                    ps = [a + b for a, b in zip(ps[0::2], ps[1::2])]
                vecs.append(ps[0])
            for m, perm in merge_consts:
                nxt = []
                for u, v in zip(vecs[0::2], vecs[1::2]):
                    up = u.at[perm].get(mode=gmode)
                    vp = v.at[perm].get(mode=gmode)
                    nxt.append(jnp.where(m, v, u) + jnp.where(m, vp, up))
                vecs = nxt
            res = vecs[0]
            pred = 1.0 / (1.0 + jnp.exp(-res))
            out_v[pl.ds(c * CH + row0, 16)] = pred
            return 0

        lax.fori_loop(0, GRP, group_body, 0)

    pltpu.sync_copy(out_v, out_hbm.at[pl.ds(base, BPW)])


@jax.jit
def kernel(x, W, H):
    xf = x.reshape(2 * B)
    mesh = plsc.VectorSubcoreMesh(core_axis_name="c", subcore_axis_name="s")
    f = pl.kernel(
        _mf_body,
        out_type=jax.ShapeDtypeStruct((B,), jnp.float32),
        mesh=mesh,
        scratch_types=[
            pltpu.VMEM((2 * BPW,), jnp.int32),
            pltpu.VMEM((NCH, CH), jnp.int32),
            pltpu.VMEM((NCH, CH), jnp.int32),
            pltpu.VMEM((CH, EMB), jnp.float32),
            pltpu.VMEM((CH, EMB), jnp.float32),
            pltpu.VMEM((CH, EMB), jnp.float32),
            pltpu.VMEM((CH, EMB), jnp.float32),
            pltpu.VMEM((BPW,), jnp.float32),
            pltpu.SemaphoreType.DMA,
            pltpu.SemaphoreType.DMA,
            pltpu.SemaphoreType.DMA,
            pltpu.SemaphoreType.DMA,
        ],
    )
    return f(xf, W, H)


# speedup vs baseline: 1.2370x; 1.2370x over previous
import jax
import jax.numpy as jnp
from jax import lax
from jax.experimental import pallas as pl
from jax.experimental.pallas import tpu as pltpu
from jax.experimental.pallas import tpu_sc as plsc

B = 16384
EMB = 128
NC = 2
NS = 16
NW = NC * NS
BPW = B // NW
CH = 128
NCH = BPW // CH
GRP = CH // 16


def _mf_body(x_hbm, w_hbm, h_hbm, out_hbm,
             xv, uid_v, iid_v, wb0, wb1, hb0, hb1, out_v,
             sw0, sw1, sh0, sh1):
    wid = lax.axis_index("s") * NC + lax.axis_index("c")
    base = wid * BPW

    lane = lax.iota(jnp.int32, 16)
    gmode = "promise_in_bounds"

    pltpu.sync_copy(x_hbm.at[pl.ds(base * 2, 2 * BPW)], xv)
    perm_u = (2 * lane) & 15
    perm_i = perm_u + 1
    lo_half = lane < 8
    for c in range(NCH):
        for g in range(GRP):
            off = 2 * (c * CH + g * 16)
            a = xv[pl.ds(off, 16)]
            b = xv[pl.ds(off + 16, 16)]
            au = a.at[perm_u].get(mode=gmode)
            bu = b.at[perm_u].get(mode=gmode)
            ai = a.at[perm_i].get(mode=gmode)
            bi = b.at[perm_i].get(mode=gmode)
            uid_v[c, pl.ds(g * 16, 16)] = jnp.where(lo_half, au, bu)
            iid_v[c, pl.ds(g * 16, 16)] = jnp.where(lo_half, ai, bi)

    wbufs = (wb0, wb1)
    hbufs = (hb0, hb1)
    wsems = (sw0, sw1)
    hsems = (sh0, sh1)

    def start(c):
        slot = c % 2
        cw = pltpu.async_copy(w_hbm.at[uid_v.at[c]], wbufs[slot], wsems[slot])
        chh = pltpu.async_copy(h_hbm.at[iid_v.at[c]], hbufs[slot], hsems[slot])
        return cw, chh

    butterfly_perms = [lane ^ s for s in (8, 4, 2, 1)]

    inflight = {0: start(0)}
    for c in range(NCH):
        if c + 1 < NCH:
            inflight[c + 1] = start(c + 1)
        for h in inflight.pop(c):
            h.wait()
        slot = c % 2
        wref = wbufs[slot]
        href = hbufs[slot]

        def group_body(g, _, wref=wref, href=href, c=c):
            row0 = g * 16

            def row_body(r, res):
                row = row0 + r
                ps = []
                for j in range(EMB // 16):
                    w = wref[row, pl.ds(j * 16, 16)]
                    h = href[row, pl.ds(j * 16, 16)]
                    ps.append(w * h)
                while len(ps) > 1:
                    ps = [a + b for a, b in zip(ps[0::2], ps[1::2])]
                acc = ps[0]
                for perm in butterfly_perms:
                    acc = acc + acc.at[perm].get(mode=gmode)
                return jnp.where(lane == r, acc, res)

            res = lax.fori_loop(0, 16, row_body, jnp.zeros((16,), jnp.float32))
            pred = 1.0 / (1.0 + jnp.exp(-res))
            out_v[pl.ds(c * CH + row0, 16)] = pred
            return 0

        lax.fori_loop(0, GRP, group_body, 0)

    pltpu.sync_copy(out_v, out_hbm.at[pl.ds(base, BPW)])


@jax.jit
def kernel(x, W, H):
    xf = x.reshape(2 * B)
    mesh = plsc.VectorSubcoreMesh(core_axis_name="c", subcore_axis_name="s")
    f = pl.kernel(
        _mf_body,
        out_type=jax.ShapeDtypeStruct((B,), jnp.float32),
        mesh=mesh,
        scratch_types=[
            pltpu.VMEM((2 * BPW,), jnp.int32),
            pltpu.VMEM((NCH, CH), jnp.int32),
            pltpu.VMEM((NCH, CH), jnp.int32),
            pltpu.VMEM((CH, EMB), jnp.float32),
            pltpu.VMEM((CH, EMB), jnp.float32),
            pltpu.VMEM((CH, EMB), jnp.float32),
            pltpu.VMEM((CH, EMB), jnp.float32),
            pltpu.VMEM((BPW,), jnp.float32),
            pltpu.SemaphoreType.DMA,
            pltpu.SemaphoreType.DMA,
            pltpu.SemaphoreType.DMA,
            pltpu.SemaphoreType.DMA,
        ],
    )
    return f(xf, W, H)


# speedup vs baseline: 1.2447x; 1.0062x over previous
import jax
import jax.numpy as jnp
from jax import lax
from jax.experimental import pallas as pl
from jax.experimental.pallas import tpu as pltpu
from jax.experimental.pallas import tpu_sc as plsc

B = 16384
EMB = 128
NC = 2
NS = 16
NW = NC * NS
BPW = B // NW
CH = 128
NCH = BPW // CH
GRP = CH // 16


def _mf_body(x_hbm, w_hbm, h_hbm, out_hbm,
             xv, uid_v, iid_v, wb0, wb1, hb0, hb1, out_v,
             sw0, sw1, sh0, sh1):
    wid = lax.axis_index("s") * NC + lax.axis_index("c")
    base = wid * BPW

    lane = lax.iota(jnp.int32, 16)
    gmode = "promise_in_bounds"

    pltpu.sync_copy(x_hbm.at[pl.ds(base * 2, 2 * BPW)], xv)
    perm_u = (2 * lane) & 15
    perm_i = perm_u + 1
    lo_half = lane < 8

    def deint(gi, _):
        off = 32 * gi
        a = xv[pl.ds(off, 16)]
        b = xv[pl.ds(off + 16, 16)]
        au = a.at[perm_u].get(mode=gmode)
        bu = b.at[perm_u].get(mode=gmode)
        ai = a.at[perm_i].get(mode=gmode)
        bi = b.at[perm_i].get(mode=gmode)
        uid_v[pl.ds(gi * 16, 16)] = jnp.where(lo_half, au, bu)
        iid_v[pl.ds(gi * 16, 16)] = jnp.where(lo_half, ai, bi)
        return 0

    wbufs = (wb0, wb1)
    hbufs = (hb0, hb1)
    wsems = (sw0, sw1)
    hsems = (sh0, sh1)

    def start(c):
        slot = c % 2
        cw = pltpu.async_copy(
            w_hbm.at[uid_v.at[pl.ds(c * CH, CH)]], wbufs[slot], wsems[slot])
        chh = pltpu.async_copy(
            h_hbm.at[iid_v.at[pl.ds(c * CH, CH)]], hbufs[slot], hsems[slot])
        return cw, chh

    butterfly_perms = [lane ^ s for s in (8, 4, 2, 1)]

    lax.fori_loop(0, GRP, deint, 0)
    inflight = {0: start(0)}
    lax.fori_loop(GRP, NCH * GRP, deint, 0)
    for c in range(NCH):
        if c + 1 < NCH:
            inflight[c + 1] = start(c + 1)
        for h in inflight.pop(c):
            h.wait()
        slot = c % 2
        wref = wbufs[slot]
        href = hbufs[slot]

        def group_body(g, _, wref=wref, href=href, c=c):
            row0 = g * 16

            def row_body(r, res):
                row = row0 + r
                ps = []
                for j in range(EMB // 16):
                    w = wref[row, pl.ds(j * 16, 16)]
                    h = href[row, pl.ds(j * 16, 16)]
                    ps.append(w * h)
                while len(ps) > 1:
                    ps = [a + b for a, b in zip(ps[0::2], ps[1::2])]
                acc = ps[0]
                for perm in butterfly_perms:
                    acc = acc + acc.at[perm].get(mode=gmode)
                return jnp.where(lane == r, acc, res)

            res = lax.fori_loop(0, 16, row_body, jnp.zeros((16,), jnp.float32))
            pred = 1.0 / (1.0 + jnp.exp(-res))
            out_v[pl.ds(c * CH + row0, 16)] = pred
            return 0

        lax.fori_loop(0, GRP, group_body, 0)

    pltpu.sync_copy(out_v, out_hbm.at[pl.ds(base, BPW)])


@jax.jit
def kernel(x, W, H):
    xf = x.reshape(2 * B)
    mesh = plsc.VectorSubcoreMesh(core_axis_name="c", subcore_axis_name="s")
    f = pl.kernel(
        _mf_body,
        out_type=jax.ShapeDtypeStruct((B,), jnp.float32),
        mesh=mesh,
        scratch_types=[
            pltpu.VMEM((2 * BPW,), jnp.int32),
            pltpu.VMEM((BPW,), jnp.int32),
            pltpu.VMEM((BPW,), jnp.int32),
            pltpu.VMEM((CH, EMB), jnp.float32),
            pltpu.VMEM((CH, EMB), jnp.float32),
            pltpu.VMEM((CH, EMB), jnp.float32),
            pltpu.VMEM((CH, EMB), jnp.float32),
            pltpu.VMEM((BPW,), jnp.float32),
            pltpu.SemaphoreType.DMA,
            pltpu.SemaphoreType.DMA,
            pltpu.SemaphoreType.DMA,
            pltpu.SemaphoreType.DMA,
        ],
    )
    return f(xf, W, H)
